# baseline (speedup 1.0000x reference)
import jax
import jax.numpy as jnp
from jax import lax
from jax.experimental import pallas as pl
from jax.experimental.pallas import tpu as pltpu
from jax.experimental.pallas import tpu_sc as plsc

VOCAB = 100000
DIM = 128
PAD = 1
B = 1024
L = 200
EPS = 1e-5

NC = 2
NS = 16
NW = NC * NS
ROWS_PER_W = B // NW
LP = 208
NCHUNK = LP // 16
NPOS = L + 2
KD = DIM // 16

_MAGIC = 0x5F3759DF
_INV_D = 1.0 / DIM


def _sc_body(ids_hbm, word_hbm, aux_hbm, gamma_hbm, beta_hbm, out_hbm,
             ids_v, idx_a0, idx_b0, idx_a1, idx_b1,
             pdx_a0, pdx_b0, pdx_a1, pdx_b1, wbuf0, wbuf1, pbuf0, pbuf1,
             gv, bv, sem_ga0, sem_gb0, sem_ga1, sem_gb1,
             sem_pa0, sem_pb0, sem_pa1, sem_pb1, sem_o0, sem_o1):
    wid = lax.axis_index("s") * NC + lax.axis_index("c")
    lane = lax.iota(jnp.int32, 16)

    pltpu.sync_copy(gamma_hbm, gv)
    pltpu.sync_copy(beta_hbm, bv)
    g = [gv[pl.ds(k * 16, 16)] for k in range(KD)]
    b = [bv[pl.ds(k * 16, 16)] for k in range(KD)]

    bufs = (
        (idx_a0, idx_b0, pdx_a0, pdx_b0, wbuf0, pbuf0,
         sem_ga0, sem_gb0, sem_pa0, sem_pb0, sem_o0),
        (idx_a1, idx_b1, pdx_a1, pdx_b1, wbuf1, pbuf1,
         sem_ga1, sem_gb1, sem_pa1, sem_pb1, sem_o1),
    )

    def prep(r, bi):
        (idx_a, idx_b, pdx_a, pdx_b, wbuf_v, pbuf_v,
         sem_ga, sem_gb, sem_pa, sem_pb, _) = bufs[bi]
        rb = wid * ROWS_PER_W + r
        pltpu.sync_copy(ids_hbm.at[pl.ds(rb * L, L)], ids_v.at[pl.ds(0, L)])
        tail = ids_v[pl.ds(192, 16)]
        ids_v[pl.ds(192, 16)] = jnp.where(lane < 8, tail, PAD)

        carry = jnp.int32(0)
        for c in range(NCHUNK):
            iv = ids_v[pl.ds(c * 16, 16)]
            m = (iv != PAD).astype(jnp.int32)
            s = jnp.cumsum(m)
            pos = jnp.where(iv != PAD, s + carry + PAD, NPOS)
            if c < 8:
                idx_a[pl.ds(c * 16, 16)] = iv
                pdx_a[pl.ds(c * 16, 16)] = pos
            else:
                idx_b[pl.ds((c - 8) * 16, 16)] = iv
                pdx_b[pl.ds((c - 8) * 16, 16)] = pos
            carry = carry + jnp.sum(m)

        pltpu.async_copy(word_hbm.at[idx_a], wbuf_v.at[pl.ds(0, 128)],
                         sem_ga)
        pltpu.async_copy(word_hbm.at[idx_b], wbuf_v.at[pl.ds(128, 80)],
                         sem_gb)
        pltpu.async_copy(aux_hbm.at[pdx_a], pbuf_v.at[pl.ds(0, 128)],
                         sem_pa)
        pltpu.async_copy(aux_hbm.at[pdx_b], pbuf_v.at[pl.ds(128, 80)],
                         sem_pb)

    def wait_gather(bi):
        (idx_a, idx_b, pdx_a, pdx_b, wbuf_v, pbuf_v,
         sem_ga, sem_gb, sem_pa, sem_pb, _) = bufs[bi]
        pltpu.make_async_copy(word_hbm.at[idx_a], wbuf_v.at[pl.ds(0, 128)],
                              sem_ga).wait()
        pltpu.make_async_copy(word_hbm.at[idx_b], wbuf_v.at[pl.ds(128, 80)],
                              sem_gb).wait()
        pltpu.make_async_copy(aux_hbm.at[pdx_a], pbuf_v.at[pl.ds(0, 128)],
                              sem_pa).wait()
        pltpu.make_async_copy(aux_hbm.at[pdx_b], pbuf_v.at[pl.ds(128, 80)],
                              sem_pb).wait()

    def fire_out(r, bi):
        wbuf_v, sem_o = bufs[bi][4], bufs[bi][10]
        rb = wid * ROWS_PER_W + r
        pltpu.async_copy(wbuf_v.at[pl.ds(0, L)],
                         out_hbm.at[pl.ds(rb * L, L)], sem_o)

    def wait_out(bi):
        wbuf_v, sem_o = bufs[bi][4], bufs[bi][10]
        pltpu.make_async_copy(wbuf_v.at[pl.ds(0, L)],
                              out_hbm.at[pl.ds(0, L)], sem_o).wait()

    def token(c, j, bi):
        wbuf_v, pbuf_v = bufs[bi][4], bufs[bi][5]
        t = c * 16 + j
        e = []
        for k in range(KD):
            w = wbuf_v[t, pl.ds(k * 16, 16)]
            p = pbuf_v[t, pl.ds(k * 16, 16)]
            e.append(w + p)
        s01 = (e[0] + e[1]) + (e[2] + e[3])
        s23 = (e[4] + e[5]) + (e[6] + e[7])
        tot = jnp.broadcast_to(jnp.sum(s01 + s23), (16,))
        q01 = (e[0] * e[0] + e[1] * e[1]) + (e[2] * e[2] + e[3] * e[3])
        q23 = (e[4] * e[4] + e[5] * e[5]) + (e[6] * e[6] + e[7] * e[7])
        totq = jnp.broadcast_to(jnp.sum(q01 + q23), (16,))
        mu = tot * _INV_D
        var = totq * _INV_D - mu * mu
        x = var + EPS
        i = lax.bitcast_convert_type(x, jnp.int32)
        y = lax.bitcast_convert_type(
            jnp.int32(_MAGIC) - lax.shift_right_arithmetic(i, 1),
            jnp.float32)
        y = y * (1.5 - 0.5 * x * y * y)
        y = y * (1.5 - 0.5 * x * y * y)
        y = y * (1.5 - 0.5 * x * y * y)
        c2 = -mu * y
        for k in range(KD):
            o = e[k] * y + c2
            wbuf_v[t, pl.ds(k * 16, 16)] = o * g[k] + b[k]

    def compute(bi):
        def chunk_body(c, carry1):
            for j in range(16):
                token(c, j, bi)
            return carry1

        lax.fori_loop(0, NCHUNK - 1, chunk_body, 0)
        for j in range(16):
            token(NCHUNK - 1, j, bi)

    prep(0, 0)

    def pair_body(i, carry0):
        r0 = 2 * i

        @pl.when(i > 0)
        def _():
            wait_out(1)

        prep(r0 + 1, 1)
        wait_gather(0)
        compute(0)
        fire_out(r0, 0)

        @pl.when(i < ROWS_PER_W // 2 - 1)
        def _():
            wait_out(0)
            prep(r0 + 2, 0)

        wait_gather(1)
        compute(1)
        fire_out(r0 + 1, 1)
        return carry0

    lax.fori_loop(0, ROWS_PER_W // 2, pair_body, 0)
    wait_out(0)
    wait_out(1)


@jax.jit
def _run(input_ids, word_emb, pos_emb, gamma, beta):
    aux = jnp.concatenate(
        [pos_emb[:NPOS], (pos_emb[PAD] - word_emb[PAD])[None]], axis=0)
    mesh = plsc.VectorSubcoreMesh(core_axis_name="c", subcore_axis_name="s")
    f = pl.kernel(
        _sc_body,
        out_type=jax.ShapeDtypeStruct((B * L, DIM), jnp.float32),
        mesh=mesh,
        scratch_types=[
            pltpu.VMEM((LP,), jnp.int32),
            pltpu.VMEM((128,), jnp.int32),
            pltpu.VMEM((80,), jnp.int32),
            pltpu.VMEM((128,), jnp.int32),
            pltpu.VMEM((80,), jnp.int32),
            pltpu.VMEM((128,), jnp.int32),
            pltpu.VMEM((80,), jnp.int32),
            pltpu.VMEM((128,), jnp.int32),
            pltpu.VMEM((80,), jnp.int32),
            pltpu.VMEM((LP, DIM), jnp.float32),
            pltpu.VMEM((LP, DIM), jnp.float32),
            pltpu.VMEM((LP, DIM), jnp.float32),
            pltpu.VMEM((LP, DIM), jnp.float32),
            pltpu.VMEM((DIM,), jnp.float32),
            pltpu.VMEM((DIM,), jnp.float32),
            pltpu.SemaphoreType.DMA,
            pltpu.SemaphoreType.DMA,
            pltpu.SemaphoreType.DMA,
            pltpu.SemaphoreType.DMA,
            pltpu.SemaphoreType.DMA,
            pltpu.SemaphoreType.DMA,
            pltpu.SemaphoreType.DMA,
            pltpu.SemaphoreType.DMA,
            pltpu.SemaphoreType.DMA,
            pltpu.SemaphoreType.DMA,
        ],
        compiler_params=pltpu.CompilerParams(needs_layout_passes=False),
    )
    return f(input_ids, word_emb, aux, gamma, beta)


def kernel(input_ids, word_emb, pos_emb, gamma, beta):
    out = _run(input_ids.astype(jnp.int32).reshape(-1), word_emb,
               pos_emb, gamma, beta)
    return out.reshape(B, L, DIM)


# speedup vs baseline: 1.0540x; 1.0540x over previous
import jax
import jax.numpy as jnp
from jax import lax
from jax.experimental import pallas as pl
from jax.experimental.pallas import tpu as pltpu
from jax.experimental.pallas import tpu_sc as plsc

VOCAB = 100000
DIM = 128
PAD = 1
B = 1024
L = 200
EPS = 1e-5

NC = 2
NS = 16
NW = NC * NS
ROWS_PER_W = B // NW
LP = 208
NCHUNK = LP // 16
NPOS = L + 2
KD = DIM // 16

_MAGIC = 0x5F3759DF
_INV_D = 1.0 / DIM


def _sc_body(ids_hbm, word_hbm, aux_hbm, gamma_hbm, beta_hbm, out_hbm,
             ids_v, idx_a0, idx_b0, idx_a1, idx_b1, posid0, posid1,
             postab_v, wbuf0, wbuf1, ebuf_v, cbuf1_v, cbuf2_v, gv, bv,
             sem_ga0, sem_gb0, sem_ga1, sem_gb1, sem_o0, sem_o1):
    wid = lax.axis_index("s") * NC + lax.axis_index("c")
    lane = lax.iota(jnp.int32, 16)

    pltpu.sync_copy(aux_hbm, postab_v)
    pltpu.sync_copy(gamma_hbm, gv)
    pltpu.sync_copy(beta_hbm, bv)
    g = [gv[pl.ds(k * 16, 16)] for k in range(KD)]
    b = [bv[pl.ds(k * 16, 16)] for k in range(KD)]

    bufs = (
        (idx_a0, idx_b0, posid0, wbuf0, sem_ga0, sem_gb0, sem_o0),
        (idx_a1, idx_b1, posid1, wbuf1, sem_ga1, sem_gb1, sem_o1),
    )

    def prep(r, bi):
        idx_a, idx_b, posid_v, wbuf_v, sem_ga, sem_gb, _ = bufs[bi]
        rb = wid * ROWS_PER_W + r
        pltpu.sync_copy(ids_hbm.at[pl.ds(rb * L, L)], ids_v.at[pl.ds(0, L)])
        tail = ids_v[pl.ds(192, 16)]
        ids_v[pl.ds(192, 16)] = jnp.where(lane < 8, tail, PAD)

        carry = jnp.int32(0)
        for c in range(NCHUNK):
            iv = ids_v[pl.ds(c * 16, 16)]
            if c < 8:
                idx_a[pl.ds(c * 16, 16)] = iv
            else:
                idx_b[pl.ds((c - 8) * 16, 16)] = iv
            m = (iv != PAD).astype(jnp.int32)
            s = jnp.cumsum(m)
            posid_v[pl.ds(c * 16, 16)] = jnp.where(iv != PAD,
                                                   s + carry + PAD, NPOS)
            carry = carry + jnp.sum(m)

        pltpu.async_copy(word_hbm.at[idx_a], wbuf_v.at[pl.ds(0, 128)],
                         sem_ga)
        pltpu.async_copy(word_hbm.at[idx_b], wbuf_v.at[pl.ds(128, 80)],
                         sem_gb)

    def wait_gather(bi):
        idx_a, idx_b, _, wbuf_v, sem_ga, sem_gb, _ = bufs[bi]
        pltpu.make_async_copy(word_hbm.at[idx_a], wbuf_v.at[pl.ds(0, 128)],
                              sem_ga).wait()
        pltpu.make_async_copy(word_hbm.at[idx_b], wbuf_v.at[pl.ds(128, 80)],
                              sem_gb).wait()

    def fire_out(r, bi):
        wbuf_v, sem_o = bufs[bi][3], bufs[bi][6]
        rb = wid * ROWS_PER_W + r
        pltpu.async_copy(wbuf_v.at[pl.ds(0, L)],
                         out_hbm.at[pl.ds(rb * L, L)], sem_o)

    def wait_out(bi):
        wbuf_v, sem_o = bufs[bi][3], bufs[bi][6]
        pltpu.make_async_copy(wbuf_v.at[pl.ds(0, L)],
                              out_hbm.at[pl.ds(0, L)], sem_o).wait()

    def compute(bi):
        _, _, posid_v, wbuf_v, _, _, _ = bufs[bi]

        def chunk_body(c, carry1):
            pidv = posid_v[pl.ds(c * 16, 16)]
            zf = lane * 0.0
            tot_v = zf
            totq_v = zf
            for j in range(16):
                t = c * 16 + j
                onehot = lane == j
                pid = jnp.sum(jnp.where(onehot, pidv, 0))
                base = pid * DIM
                e = []
                for k in range(KD):
                    w = wbuf_v[t, pl.ds(k * 16, 16)]
                    p = postab_v[pl.ds(base + k * 16, 16)]
                    ek = w + p
                    ebuf_v[pl.ds(t * DIM + k * 16, 16)] = ek
                    e.append(ek)
                s01 = (e[0] + e[1]) + (e[2] + e[3])
                s23 = (e[4] + e[5]) + (e[6] + e[7])
                q01 = (e[0] * e[0] + e[1] * e[1]) + \
                    (e[2] * e[2] + e[3] * e[3])
                q23 = (e[4] * e[4] + e[5] * e[5]) + \
                    (e[6] * e[6] + e[7] * e[7])
                tot_v = jnp.where(onehot, jnp.sum(s01 + s23), tot_v)
                totq_v = jnp.where(onehot, jnp.sum(q01 + q23), totq_v)
            mu = tot_v * _INV_D
            var = totq_v * _INV_D - mu * mu
            x = var + EPS
            i = lax.bitcast_convert_type(x, jnp.int32)
            y = lax.bitcast_convert_type(
                jnp.int32(_MAGIC) - lax.shift_right_arithmetic(i, 1),
                jnp.float32)
            y = y * (1.5 - 0.5 * x * y * y)
            y = y * (1.5 - 0.5 * x * y * y)
            y = y * (1.5 - 0.5 * x * y * y)
            c2 = -mu * y
            for col in range(16):
                plsc.store_scatter(cbuf1_v, [lane * 16 + col], y)
                plsc.store_scatter(cbuf2_v, [lane * 16 + col], c2)
            for j in range(16):
                t = c * 16 + j
                c1j = cbuf1_v[pl.ds(j * 16, 16)]
                c2j = cbuf2_v[pl.ds(j * 16, 16)]
                for k in range(KD):
                    ek = ebuf_v[pl.ds(t * DIM + k * 16, 16)]
                    o = ek * c1j + c2j
                    wbuf_v[t, pl.ds(k * 16, 16)] = o * g[k] + b[k]
            return carry1

        lax.fori_loop(0, NCHUNK, chunk_body, 0)

    prep(0, 0)

    def pair_body(i, carry0):
        r0 = 2 * i

        @pl.when(i > 0)
        def _():
            wait_out(1)

        prep(r0 + 1, 1)
        wait_gather(0)
        compute(0)
        fire_out(r0, 0)

        @pl.when(i < ROWS_PER_W // 2 - 1)
        def _():
            wait_out(0)
            prep(r0 + 2, 0)

        wait_gather(1)
        compute(1)
        fire_out(r0 + 1, 1)
        return carry0

    lax.fori_loop(0, ROWS_PER_W // 2, pair_body, 0)
    wait_out(0)
    wait_out(1)


@jax.jit
def _run(input_ids, word_emb, pos_emb, gamma, beta):
    aux = jnp.concatenate(
        [pos_emb[:NPOS], (pos_emb[PAD] - word_emb[PAD])[None]],
        axis=0).reshape(-1)
    mesh = plsc.VectorSubcoreMesh(core_axis_name="c", subcore_axis_name="s")
    f = pl.kernel(
        _sc_body,
        out_type=jax.ShapeDtypeStruct((B * L, DIM), jnp.float32),
        mesh=mesh,
        scratch_types=[
            pltpu.VMEM((LP,), jnp.int32),
            pltpu.VMEM((128,), jnp.int32),
            pltpu.VMEM((80,), jnp.int32),
            pltpu.VMEM((128,), jnp.int32),
            pltpu.VMEM((80,), jnp.int32),
            pltpu.VMEM((LP,), jnp.int32),
            pltpu.VMEM((LP,), jnp.int32),
            pltpu.VMEM(((NPOS + 1) * DIM,), jnp.float32),
            pltpu.VMEM((LP, DIM), jnp.float32),
            pltpu.VMEM((LP, DIM), jnp.float32),
            pltpu.VMEM((LP * DIM,), jnp.float32),
            pltpu.VMEM((256,), jnp.float32),
            pltpu.VMEM((256,), jnp.float32),
            pltpu.VMEM((DIM,), jnp.float32),
            pltpu.VMEM((DIM,), jnp.float32),
            pltpu.SemaphoreType.DMA,
            pltpu.SemaphoreType.DMA,
            pltpu.SemaphoreType.DMA,
            pltpu.SemaphoreType.DMA,
            pltpu.SemaphoreType.DMA,
            pltpu.SemaphoreType.DMA,
        ],
        compiler_params=pltpu.CompilerParams(needs_layout_passes=False),
    )
    return f(input_ids, word_emb, aux, gamma, beta)


def kernel(input_ids, word_emb, pos_emb, gamma, beta):
    out = _run(input_ids.astype(jnp.int32).reshape(-1), word_emb,
               pos_emb, gamma, beta)
    return out.reshape(B, L, DIM)


# speedup vs baseline: 1.3504x; 1.2813x over previous
import jax
import jax.numpy as jnp
from jax import lax
from jax.experimental import pallas as pl
from jax.experimental.pallas import tpu as pltpu
from jax.experimental.pallas import tpu_sc as plsc

VOCAB = 100000
DIM = 128
PAD = 1
B = 1024
L = 200
EPS = 1e-5

NC = 2
NS = 16
NW = NC * NS
ROWS_PER_W = B // NW
NCHUNK = L // 16
NPOS = L + 2
KD = DIM // 16

_MAGIC = 0x5F3759DF
_INV_D = 1.0 / DIM


N_A = 96 * DIM
N_B = 104 * DIM


def _sc_body(ids_hbm, word_hbm, aux_hbm, gamma_hbm, beta_hbm, out_hbm,
             ids0, ids1, idx_a0, idx_b0, idx_a1, idx_b1, posid0, posid1,
             postab_v, wbuf0, wbuf1, obuf_v, cbuf1_v, cbuf2_v, gv, bv,
             sem_i0, sem_i1, sem_ga0, sem_gb0, sem_ga1, sem_gb1,
             sem_oa, sem_ob):
    wid = lax.axis_index("s") * NC + lax.axis_index("c")
    lane = lax.iota(jnp.int32, 16)

    pltpu.sync_copy(aux_hbm.at[pl.ds(DIM, NPOS * DIM)], postab_v)
    pltpu.sync_copy(gamma_hbm, gv)
    pltpu.sync_copy(beta_hbm, bv)

    ids = (ids0, ids1)
    sem_i = (sem_i0, sem_i1)
    ibufs = (
        (idx_a0, idx_b0, posid0, wbuf0, sem_ga0, sem_gb0),
        (idx_a1, idx_b1, posid1, wbuf1, sem_ga1, sem_gb1),
    )

    def ids_copy(r, bi):
        rb = wid * ROWS_PER_W + r
        return pltpu.make_async_copy(ids_hbm.at[pl.ds(rb * L, L)],
                                     ids[bi].at[pl.ds(0, L)], sem_i[bi])

    def fire_ids(r, bi):
        rb = wid * ROWS_PER_W + r
        pltpu.async_copy(ids_hbm.at[pl.ds(rb * L, L)],
                         ids[bi].at[pl.ds(0, L)], sem_i[bi])

    def prep(r, bi):
        idx_a, idx_b, posid_v, wbuf_v, sem_ga, sem_gb = ibufs[bi]
        ids_v = ids[bi]
        tail = ids_v[pl.ds(192, 16)]
        ids_v[pl.ds(192, 16)] = jnp.where(lane < 8, tail, PAD)

        carry = jnp.int32(0)
        for c in range(NCHUNK + 1):
            iv = ids_v[pl.ds(c * 16, 16)]
            if c < 8:
                idx_a[pl.ds(c * 16, 16)] = iv
            else:
                idx_b[pl.ds((c - 8) * 16, 16)] = iv
            m = (iv != PAD).astype(jnp.int32)
            s = jnp.cumsum(m)
            posid_v[pl.ds(c * 16, 16)] = jnp.where(iv != PAD,
                                                   s + carry, NPOS - 1)
            carry = carry + jnp.sum(m)

        pltpu.async_copy(word_hbm.at[idx_a], wbuf_v.at[pl.ds(0, 128)],
                         sem_ga)
        pltpu.async_copy(word_hbm.at[idx_b.at[pl.ds(0, 72)]],
                         wbuf_v.at[pl.ds(128, 72)], sem_gb)

    def wait_gather(bi):
        idx_a, idx_b, _, wbuf_v, sem_ga, sem_gb = ibufs[bi]
        pltpu.make_async_copy(word_hbm.at[idx_a], wbuf_v.at[pl.ds(0, 128)],
                              sem_ga).wait()
        pltpu.make_async_copy(word_hbm.at[idx_b.at[pl.ds(0, 72)]],
                              wbuf_v.at[pl.ds(128, 72)], sem_gb).wait()

    def fire_half_a(r):
        rb = wid * ROWS_PER_W + r
        pltpu.async_copy(obuf_v.at[pl.ds(0, N_A)],
                         out_hbm.at[pl.ds(rb * L * DIM, N_A)], sem_oa)

    def fire_half_b(r):
        rb = wid * ROWS_PER_W + r
        pltpu.async_copy(obuf_v.at[pl.ds(N_A, N_B)],
                         out_hbm.at[pl.ds(rb * L * DIM + N_A, N_B)], sem_ob)

    def wait_half_a():
        pltpu.make_async_copy(obuf_v.at[pl.ds(0, N_A)],
                              out_hbm.at[pl.ds(0, N_A)], sem_oa).wait()

    def wait_half_b():
        pltpu.make_async_copy(obuf_v.at[pl.ds(N_A, N_B)],
                              out_hbm.at[pl.ds(0, N_B)], sem_ob).wait()

    def compute(bi, r):
        _, _, posid_v, wbuf_v, _, _ = ibufs[bi]

        def do_chunk(c, nj):
            pidv = posid_v[pl.ds(c * 16, 16)]
            zf = lane * 0.0
            tot_v = zf
            totq_v = zf
            for j in range(nj):
                t = c * 16 + j
                onehot = lane == j
                pid = jnp.sum(jnp.where(onehot, pidv, 0))
                base = pid * DIM
                e = []
                for k in range(KD):
                    w = wbuf_v[t, pl.ds(k * 16, 16)]
                    p = postab_v[pl.ds(base + k * 16, 16)]
                    ek = w + p
                    wbuf_v[t, pl.ds(k * 16, 16)] = ek
                    e.append(ek)
                s01 = (e[0] + e[1]) + (e[2] + e[3])
                s23 = (e[4] + e[5]) + (e[6] + e[7])
                q01 = (e[0] * e[0] + e[1] * e[1]) + \
                    (e[2] * e[2] + e[3] * e[3])
                q23 = (e[4] * e[4] + e[5] * e[5]) + \
                    (e[6] * e[6] + e[7] * e[7])
                tot_v = jnp.where(onehot, jnp.sum(s01 + s23), tot_v)
                totq_v = jnp.where(onehot, jnp.sum(q01 + q23), totq_v)
            mu = tot_v * _INV_D
            var = totq_v * _INV_D - mu * mu
            x = var + EPS
            i = lax.bitcast_convert_type(x, jnp.int32)
            y = lax.bitcast_convert_type(
                jnp.int32(_MAGIC) - lax.shift_right_arithmetic(i, 1),
                jnp.float32)
            y = y * (1.5 - 0.5 * x * y * y)
            y = y * (1.5 - 0.5 * x * y * y)
            y = y * (1.5 - 0.5 * x * y * y)
            c2 = -mu * y
            for col in range(16):
                plsc.store_scatter(cbuf1_v, [lane * 16 + col], y)
                plsc.store_scatter(cbuf2_v, [lane * 16 + col], c2)
            g = [gv[pl.ds(k * 16, 16)] for k in range(KD)]
            b = [bv[pl.ds(k * 16, 16)] for k in range(KD)]
            for j in range(nj):
                t = c * 16 + j
                c1j = cbuf1_v[pl.ds(j * 16, 16)]
                c2j = cbuf2_v[pl.ds(j * 16, 16)]
                for k in range(KD):
                    ek = wbuf_v[t, pl.ds(k * 16, 16)]
                    o = ek * c1j + c2j
                    obuf_v[pl.ds(t * DIM + k * 16, 16)] = o * g[k] + b[k]

        def chunk_body(c, carry1):
            @pl.when((c == 6) & (r > 0))
            def _():
                wait_half_b()

            do_chunk(c, 16)

            @pl.when(c == 5)
            def _():
                fire_half_a(r)

            return carry1

        @pl.when(r > 0)
        def _():
            wait_half_a()

        lax.fori_loop(0, NCHUNK, chunk_body, 0)
        do_chunk(NCHUNK, 8)
        fire_half_b(r)

    fire_ids(0, 0)
    ids_copy(0, 0).wait()
    prep(0, 0)
    fire_ids(1, 1)

    def pair_body(i, carry0):
        r0 = 2 * i
        ids_copy(r0 + 1, 1).wait()
        prep(r0 + 1, 1)

        @pl.when(i < ROWS_PER_W // 2 - 1)
        def _():
            fire_ids(r0 + 2, 0)

        wait_gather(0)
        compute(0, r0)

        @pl.when(i < ROWS_PER_W // 2 - 1)
        def _():
            ids_copy(r0 + 2, 0).wait()
            prep(r0 + 2, 0)
            fire_ids(r0 + 3, 1)

        wait_gather(1)
        compute(1, r0 + 1)
        return carry0

    lax.fori_loop(0, ROWS_PER_W // 2, pair_body, 0)
    wait_half_a()
    wait_half_b()


@jax.jit
def _run(input_ids, word_emb, pos_emb, gamma, beta):
    aux = jnp.concatenate(
        [pos_emb[:NPOS], (pos_emb[PAD] - word_emb[PAD])[None]],
        axis=0).reshape(-1)
    mesh = plsc.VectorSubcoreMesh(core_axis_name="c", subcore_axis_name="s")
    f = pl.kernel(
        _sc_body,
        out_type=jax.ShapeDtypeStruct((B * L * DIM,), jnp.float32),
        mesh=mesh,
        scratch_types=[
            pltpu.VMEM((208,), jnp.int32),
            pltpu.VMEM((208,), jnp.int32),
            pltpu.VMEM((128,), jnp.int32),
            pltpu.VMEM((80,), jnp.int32),
            pltpu.VMEM((128,), jnp.int32),
            pltpu.VMEM((80,), jnp.int32),
            pltpu.VMEM((208,), jnp.int32),
            pltpu.VMEM((208,), jnp.int32),
            pltpu.VMEM((NPOS * DIM,), jnp.float32),
            pltpu.VMEM((L, DIM), jnp.float32),
            pltpu.VMEM((L, DIM), jnp.float32),
            pltpu.VMEM((L * DIM,), jnp.float32),
            pltpu.VMEM((256,), jnp.float32),
            pltpu.VMEM((256,), jnp.float32),
            pltpu.VMEM((DIM,), jnp.float32),
            pltpu.VMEM((DIM,), jnp.float32),
            pltpu.SemaphoreType.DMA,
            pltpu.SemaphoreType.DMA,
            pltpu.SemaphoreType.DMA,
            pltpu.SemaphoreType.DMA,
            pltpu.SemaphoreType.DMA,
            pltpu.SemaphoreType.DMA,
            pltpu.SemaphoreType.DMA,
            pltpu.SemaphoreType.DMA,
        ],
        compiler_params=pltpu.CompilerParams(needs_layout_passes=False),
    )
    return f(input_ids, word_emb, aux, gamma, beta)


def kernel(input_ids, word_emb, pos_emb, gamma, beta):
    out = _run(input_ids.astype(jnp.int32).reshape(-1), word_emb,
               pos_emb, gamma, beta)
    return out.reshape(B, L, DIM)


# speedup vs baseline: 1.5427x; 1.1424x over previous
import jax
import jax.numpy as jnp
from jax import lax
from jax.experimental import pallas as pl
from jax.experimental.pallas import tpu as pltpu
from jax.experimental.pallas import tpu_sc as plsc

VOCAB = 100000
DIM = 128
PAD = 1
B = 1024
L = 200
EPS = 1e-5

NC = 2
NS = 16
NW = NC * NS
ROWS_PER_W = B // NW
NCHUNK = L // 16
NPOS = L + 2
KD = DIM // 16

_MAGIC = 0x5F3759DF
_INV_D = 1.0 / DIM


N_A = 96 * DIM
N_B = 104 * DIM


def _sc_body(ids_hbm, word_hbm, aux_hbm, gamma_hbm, beta_hbm, out_hbm,
             ids0, ids1, idx_a0, idx_b0, idx_a1, idx_b1, posid0, posid1,
             postab_v, wbuf0, wbuf1, obuf_v, cbuf1_v, cbuf2_v, gv, bv,
             sem_i0, sem_i1, sem_ga0, sem_gb0, sem_ga1, sem_gb1,
             sem_oa, sem_ob):
    wid = lax.axis_index("s") * NC + lax.axis_index("c")
    lane = lax.iota(jnp.int32, 16)

    pltpu.sync_copy(aux_hbm.at[pl.ds(DIM, NPOS * DIM)], postab_v)
    pltpu.sync_copy(gamma_hbm, gv)
    pltpu.sync_copy(beta_hbm, bv)

    ids = (ids0, ids1)
    sem_i = (sem_i0, sem_i1)
    ibufs = (
        (idx_a0, idx_b0, posid0, wbuf0, sem_ga0, sem_gb0),
        (idx_a1, idx_b1, posid1, wbuf1, sem_ga1, sem_gb1),
    )

    def ids_copy(r, bi):
        rb = wid * ROWS_PER_W + r
        return pltpu.make_async_copy(ids_hbm.at[pl.ds(rb * L, L)],
                                     ids[bi].at[pl.ds(0, L)], sem_i[bi])

    def fire_ids(r, bi):
        rb = wid * ROWS_PER_W + r
        pltpu.async_copy(ids_hbm.at[pl.ds(rb * L, L)],
                         ids[bi].at[pl.ds(0, L)], sem_i[bi])

    def prep(r, bi):
        idx_a, idx_b, posid_v, wbuf_v, sem_ga, sem_gb = ibufs[bi]
        ids_v = ids[bi]
        tail = ids_v[pl.ds(192, 16)]
        ids_v[pl.ds(192, 16)] = jnp.where(lane < 8, tail, PAD)

        carry = jnp.int32(0)
        for c in range(NCHUNK + 1):
            iv = ids_v[pl.ds(c * 16, 16)]
            if c < 8:
                idx_a[pl.ds(c * 16, 16)] = iv
            else:
                idx_b[pl.ds((c - 8) * 16, 16)] = iv
            m = (iv != PAD).astype(jnp.int32)
            s = jnp.cumsum(m)
            posid_v[pl.ds(c * 16, 16)] = jnp.where(iv != PAD,
                                                   s + carry, NPOS - 1)
            carry = carry + jnp.sum(m)

        pltpu.async_copy(word_hbm.at[idx_a], wbuf_v.at[pl.ds(0, 128)],
                         sem_ga)
        pltpu.async_copy(word_hbm.at[idx_b.at[pl.ds(0, 72)]],
                         wbuf_v.at[pl.ds(128, 72)], sem_gb)

    def wait_gather(bi):
        idx_a, idx_b, _, wbuf_v, sem_ga, sem_gb = ibufs[bi]
        pltpu.make_async_copy(word_hbm.at[idx_a], wbuf_v.at[pl.ds(0, 128)],
                              sem_ga).wait()
        pltpu.make_async_copy(word_hbm.at[idx_b.at[pl.ds(0, 72)]],
                              wbuf_v.at[pl.ds(128, 72)], sem_gb).wait()

    def fire_half_a(r):
        rb = wid * ROWS_PER_W + r
        pltpu.async_copy(obuf_v.at[pl.ds(0, N_A)],
                         out_hbm.at[pl.ds(rb * L * DIM, N_A)], sem_oa)

    def fire_half_b(r):
        rb = wid * ROWS_PER_W + r
        pltpu.async_copy(obuf_v.at[pl.ds(N_A, N_B)],
                         out_hbm.at[pl.ds(rb * L * DIM + N_A, N_B)], sem_ob)

    def wait_half_a():
        pltpu.make_async_copy(obuf_v.at[pl.ds(0, N_A)],
                              out_hbm.at[pl.ds(0, N_A)], sem_oa).wait()

    def wait_half_b():
        pltpu.make_async_copy(obuf_v.at[pl.ds(N_A, N_B)],
                              out_hbm.at[pl.ds(0, N_B)], sem_ob).wait()

    def _lane_splat(v, j):
        return v.at[lane * 0 + j].get(mode="promise_in_bounds")

    def _lane_sum(v):
        for d in (1, 2, 4, 8):
            v = v + v.at[lane ^ d].get(mode="promise_in_bounds")
        return v

    def compute(bi, r):
        _, _, posid_v, wbuf_v, _, _ = ibufs[bi]

        def do_chunk(c, nj):
            pidv = posid_v[pl.ds(c * 16, 16)]
            g = [gv[pl.ds(k * 16, 16)] for k in range(KD)]
            b = [bv[pl.ds(k * 16, 16)] for k in range(KD)]
            for j in range(nj):
                t = c * 16 + j
                base_v = _lane_splat(pidv, j) * DIM
                e = []
                for k in range(KD):
                    w = wbuf_v[t, pl.ds(k * 16, 16)]
                    p = plsc.load_gather(postab_v,
                                         [base_v + (lane + k * 16)])
                    e.append(w + p)
                s01 = (e[0] + e[1]) + (e[2] + e[3])
                s23 = (e[4] + e[5]) + (e[6] + e[7])
                q01 = (e[0] * e[0] + e[1] * e[1]) + \
                    (e[2] * e[2] + e[3] * e[3])
                q23 = (e[4] * e[4] + e[5] * e[5]) + \
                    (e[6] * e[6] + e[7] * e[7])
                mu = _lane_sum(s01 + s23) * _INV_D
                var = _lane_sum(q01 + q23) * _INV_D - mu * mu
                x = var + EPS
                i = lax.bitcast_convert_type(x, jnp.int32)
                y = lax.bitcast_convert_type(
                    jnp.int32(_MAGIC) - lax.shift_right_arithmetic(i, 1),
                    jnp.float32)
                y = y * (1.5 - 0.5 * x * y * y)
                y = y * (1.5 - 0.5 * x * y * y)
                y = y * (1.5 - 0.5 * x * y * y)
                c2 = -mu * y
                for k in range(KD):
                    o = e[k] * y + c2
                    obuf_v[pl.ds(t * DIM + k * 16, 16)] = o * g[k] + b[k]

        def chunk_body(c, carry1):
            @pl.when((c == 6) & (r > 0))
            def _():
                wait_half_b()

            do_chunk(c, 16)

            @pl.when(c == 5)
            def _():
                fire_half_a(r)

            return carry1

        @pl.when(r > 0)
        def _():
            wait_half_a()

        lax.fori_loop(0, NCHUNK, chunk_body, 0)
        do_chunk(NCHUNK, 8)
        fire_half_b(r)

    fire_ids(0, 0)
    ids_copy(0, 0).wait()
    prep(0, 0)
    fire_ids(1, 1)

    def pair_body(i, carry0):
        r0 = 2 * i
        ids_copy(r0 + 1, 1).wait()
        prep(r0 + 1, 1)

        @pl.when(i < ROWS_PER_W // 2 - 1)
        def _():
            fire_ids(r0 + 2, 0)

        wait_gather(0)
        compute(0, r0)

        @pl.when(i < ROWS_PER_W // 2 - 1)
        def _():
            ids_copy(r0 + 2, 0).wait()
            prep(r0 + 2, 0)
            fire_ids(r0 + 3, 1)

        wait_gather(1)
        compute(1, r0 + 1)
        return carry0

    lax.fori_loop(0, ROWS_PER_W // 2, pair_body, 0)
    wait_half_a()
    wait_half_b()


@jax.jit
def _run(input_ids, word_emb, pos_emb, gamma, beta):
    aux = jnp.concatenate(
        [pos_emb[:NPOS], (pos_emb[PAD] - word_emb[PAD])[None]],
        axis=0).reshape(-1)
    mesh = plsc.VectorSubcoreMesh(core_axis_name="c", subcore_axis_name="s")
    f = pl.kernel(
        _sc_body,
        out_type=jax.ShapeDtypeStruct((B * L * DIM,), jnp.float32),
        mesh=mesh,
        scratch_types=[
            pltpu.VMEM((208,), jnp.int32),
            pltpu.VMEM((208,), jnp.int32),
            pltpu.VMEM((128,), jnp.int32),
            pltpu.VMEM((80,), jnp.int32),
            pltpu.VMEM((128,), jnp.int32),
            pltpu.VMEM((80,), jnp.int32),
            pltpu.VMEM((208,), jnp.int32),
            pltpu.VMEM((208,), jnp.int32),
            pltpu.VMEM((NPOS * DIM,), jnp.float32),
            pltpu.VMEM((L, DIM), jnp.float32),
            pltpu.VMEM((L, DIM), jnp.float32),
            pltpu.VMEM((L * DIM,), jnp.float32),
            pltpu.VMEM((256,), jnp.float32),
            pltpu.VMEM((256,), jnp.float32),
            pltpu.VMEM((DIM,), jnp.float32),
            pltpu.VMEM((DIM,), jnp.float32),
            pltpu.SemaphoreType.DMA,
            pltpu.SemaphoreType.DMA,
            pltpu.SemaphoreType.DMA,
            pltpu.SemaphoreType.DMA,
            pltpu.SemaphoreType.DMA,
            pltpu.SemaphoreType.DMA,
            pltpu.SemaphoreType.DMA,
            pltpu.SemaphoreType.DMA,
        ],
        compiler_params=pltpu.CompilerParams(needs_layout_passes=False),
    )
    return f(input_ids, word_emb, aux, gamma, beta)


def kernel(input_ids, word_emb, pos_emb, gamma, beta):
    out = _run(input_ids.astype(jnp.int32).reshape(-1), word_emb,
               pos_emb, gamma, beta)
    return out.reshape(B, L, DIM)


# speedup vs baseline: 1.6802x; 1.0891x over previous
import jax
import jax.numpy as jnp
from jax import lax
from jax.experimental import pallas as pl
from jax.experimental.pallas import tpu as pltpu
from jax.experimental.pallas import tpu_sc as plsc

VOCAB = 100000
DIM = 128
PAD = 1
B = 1024
L = 200
EPS = 1e-5

NC = 2
NS = 16
NW = NC * NS
ROWS_PER_W = B // NW
NCHUNK = L // 16
NPOS = L + 2
KD = DIM // 16

_MAGIC = 0x5F3759DF
_INV_D = 1.0 / DIM


N_A = 96 * DIM
N_B = 104 * DIM


def _sc_body(ids_hbm, word_hbm, aux_hbm, gamma_hbm, beta_hbm, out_hbm,
             ids0, ids1, idx_a0, idx_b0, idx_a1, idx_b1, posid0, posid1,
             postab_v, wbuf0, wbuf1, obuf_v, gv, bv,
             sem_i0, sem_i1, sem_ga0, sem_gb0, sem_ga1, sem_gb1,
             sem_oa, sem_ob):
    wid = lax.axis_index("s") * NC + lax.axis_index("c")
    lane = lax.iota(jnp.int32, 16)

    pltpu.sync_copy(aux_hbm.at[pl.ds(DIM, NPOS * DIM)], postab_v)
    pltpu.sync_copy(gamma_hbm, gv)
    pltpu.sync_copy(beta_hbm, bv)

    ids = (ids0, ids1)
    sem_i = (sem_i0, sem_i1)
    ibufs = (
        (idx_a0, idx_b0, posid0, wbuf0, sem_ga0, sem_gb0),
        (idx_a1, idx_b1, posid1, wbuf1, sem_ga1, sem_gb1),
    )

    def ids_copy(r, bi):
        rb = wid * ROWS_PER_W + r
        return pltpu.make_async_copy(ids_hbm.at[pl.ds(rb * L, L)],
                                     ids[bi].at[pl.ds(0, L)], sem_i[bi])

    def fire_ids(r, bi):
        rb = wid * ROWS_PER_W + r
        pltpu.async_copy(ids_hbm.at[pl.ds(rb * L, L)],
                         ids[bi].at[pl.ds(0, L)], sem_i[bi])

    def prep(r, bi):
        idx_a, idx_b, posid_v, wbuf_v, sem_ga, sem_gb = ibufs[bi]
        ids_v = ids[bi]
        tail = ids_v[pl.ds(192, 16)]
        ids_v[pl.ds(192, 16)] = jnp.where(lane < 8, tail, PAD)

        carry = jnp.int32(0)
        for c in range(NCHUNK + 1):
            iv = ids_v[pl.ds(c * 16, 16)]
            if c < 8:
                idx_a[pl.ds(c * 16, 16)] = iv
            else:
                idx_b[pl.ds((c - 8) * 16, 16)] = iv
            m = (iv != PAD).astype(jnp.int32)
            s = jnp.cumsum(m)
            posid_v[pl.ds(c * 16, 16)] = jnp.where(iv != PAD,
                                                   s + carry, NPOS - 1)
            carry = carry + jnp.sum(m)

        pltpu.async_copy(word_hbm.at[idx_a], wbuf_v.at[pl.ds(0, 128)],
                         sem_ga)
        pltpu.async_copy(word_hbm.at[idx_b.at[pl.ds(0, 72)]],
                         wbuf_v.at[pl.ds(128, 72)], sem_gb)

    def wait_gather(bi):
        idx_a, idx_b, _, wbuf_v, sem_ga, sem_gb = ibufs[bi]
        pltpu.make_async_copy(word_hbm.at[idx_a], wbuf_v.at[pl.ds(0, 128)],
                              sem_ga).wait()
        pltpu.make_async_copy(word_hbm.at[idx_b.at[pl.ds(0, 72)]],
                              wbuf_v.at[pl.ds(128, 72)], sem_gb).wait()

    def fire_half_a(r):
        rb = wid * ROWS_PER_W + r
        pltpu.async_copy(obuf_v.at[pl.ds(0, N_A)],
                         out_hbm.at[pl.ds(rb * L * DIM, N_A)], sem_oa)

    def fire_half_b(r):
        rb = wid * ROWS_PER_W + r
        pltpu.async_copy(obuf_v.at[pl.ds(N_A, N_B)],
                         out_hbm.at[pl.ds(rb * L * DIM + N_A, N_B)], sem_ob)

    def wait_half_a():
        pltpu.make_async_copy(obuf_v.at[pl.ds(0, N_A)],
                              out_hbm.at[pl.ds(0, N_A)], sem_oa).wait()

    def wait_half_b():
        pltpu.make_async_copy(obuf_v.at[pl.ds(N_A, N_B)],
                              out_hbm.at[pl.ds(0, N_B)], sem_ob).wait()

    def _lane_splat(v, j):
        return v.at[lane * 0 + j].get(mode="promise_in_bounds")

    def _lane_sum(v):
        for d in (1, 2, 4, 8):
            v = v + v.at[lane ^ d].get(mode="promise_in_bounds")
        return v

    def compute(bi, r):
        _, _, posid_v, wbuf_v, _, _ = ibufs[bi]

        def do_chunk(c, nj):
            pidv = posid_v[pl.ds(c * 16, 16)]
            g = [gv[pl.ds(k * 16, 16)] for k in range(KD)]
            b = [bv[pl.ds(k * 16, 16)] for k in range(KD)]
            for j in range(nj):
                t = c * 16 + j
                base_v = _lane_splat(pidv, j) * DIM
                e = []
                for k in range(KD):
                    w = wbuf_v[t, pl.ds(k * 16, 16)]
                    p = plsc.load_gather(postab_v,
                                         [base_v + (lane + k * 16)])
                    e.append(w + p)
                s01 = (e[0] + e[1]) + (e[2] + e[3])
                s23 = (e[4] + e[5]) + (e[6] + e[7])
                q01 = (e[0] * e[0] + e[1] * e[1]) + \
                    (e[2] * e[2] + e[3] * e[3])
                q23 = (e[4] * e[4] + e[5] * e[5]) + \
                    (e[6] * e[6] + e[7] * e[7])
                mu = _lane_sum(s01 + s23) * _INV_D
                var = _lane_sum(q01 + q23) * _INV_D - mu * mu
                x = var + EPS
                i = lax.bitcast_convert_type(x, jnp.int32)
                y = lax.bitcast_convert_type(
                    jnp.int32(_MAGIC) - lax.shift_right_arithmetic(i, 1),
                    jnp.float32)
                y = y * (1.5 - 0.5 * x * y * y)
                y = y * (1.5 - 0.5 * x * y * y)
                c2 = -mu * y
                for k in range(KD):
                    o = e[k] * y + c2
                    obuf_v[pl.ds(t * DIM + k * 16, 16)] = o * g[k] + b[k]

        def chunk_body(c, carry1):
            @pl.when((c == 6) & (r > 0))
            def _():
                wait_half_b()

            do_chunk(c, 16)

            @pl.when(c == 5)
            def _():
                fire_half_a(r)

            return carry1

        @pl.when(r > 0)
        def _():
            wait_half_a()

        lax.fori_loop(0, NCHUNK, chunk_body, 0)
        do_chunk(NCHUNK, 8)
        fire_half_b(r)

    fire_ids(0, 0)
    ids_copy(0, 0).wait()
    prep(0, 0)
    fire_ids(1, 1)

    def pair_body(i, carry0):
        r0 = 2 * i
        ids_copy(r0 + 1, 1).wait()
        prep(r0 + 1, 1)

        @pl.when(i < ROWS_PER_W // 2 - 1)
        def _():
            fire_ids(r0 + 2, 0)

        wait_gather(0)
        compute(0, r0)

        @pl.when(i < ROWS_PER_W // 2 - 1)
        def _():
            ids_copy(r0 + 2, 0).wait()
            prep(r0 + 2, 0)
            fire_ids(r0 + 3, 1)

        wait_gather(1)
        compute(1, r0 + 1)
        return carry0

    lax.fori_loop(0, ROWS_PER_W // 2, pair_body, 0)
    wait_half_a()
    wait_half_b()


@jax.jit
def _run(input_ids, word_emb, pos_emb, gamma, beta):
    aux = jnp.concatenate(
        [pos_emb[:NPOS], (pos_emb[PAD] - word_emb[PAD])[None]],
        axis=0).reshape(-1)
    mesh = plsc.VectorSubcoreMesh(core_axis_name="c", subcore_axis_name="s")
    f = pl.kernel(
        _sc_body,
        out_type=jax.ShapeDtypeStruct((B * L * DIM,), jnp.float32),
        mesh=mesh,
        scratch_types=[
            pltpu.VMEM((208,), jnp.int32),
            pltpu.VMEM((208,), jnp.int32),
            pltpu.VMEM((128,), jnp.int32),
            pltpu.VMEM((80,), jnp.int32),
            pltpu.VMEM((128,), jnp.int32),
            pltpu.VMEM((80,), jnp.int32),
            pltpu.VMEM((208,), jnp.int32),
            pltpu.VMEM((208,), jnp.int32),
            pltpu.VMEM((NPOS * DIM,), jnp.float32),
            pltpu.VMEM((L, DIM), jnp.float32),
            pltpu.VMEM((L, DIM), jnp.float32),
            pltpu.VMEM((L * DIM,), jnp.float32),
            pltpu.VMEM((DIM,), jnp.float32),
            pltpu.VMEM((DIM,), jnp.float32),
            pltpu.SemaphoreType.DMA,
            pltpu.SemaphoreType.DMA,
            pltpu.SemaphoreType.DMA,
            pltpu.SemaphoreType.DMA,
            pltpu.SemaphoreType.DMA,
            pltpu.SemaphoreType.DMA,
            pltpu.SemaphoreType.DMA,
            pltpu.SemaphoreType.DMA,
        ],
        compiler_params=pltpu.CompilerParams(needs_layout_passes=False),
    )
    return f(input_ids, word_emb, aux, gamma, beta)


def kernel(input_ids, word_emb, pos_emb, gamma, beta):
    out = _run(input_ids.astype(jnp.int32).reshape(-1), word_emb,
               pos_emb, gamma, beta)
    return out.reshape(B, L, DIM)
